# Initial kernel scaffold; baseline (speedup 1.0000x reference)
import jax
import jax.numpy as jnp
from jax.experimental import pallas as pl


def kernel(h, m, rbf, id_j, W_rbf, W1, res_Ws):
    raise NotImplementedError("write your pallas kernel here")


# speedup vs baseline: 2.1162x; 2.1162x over previous
import functools

import jax
import jax.numpy as jnp
from jax import lax
from jax.experimental import pallas as pl
from jax.experimental.pallas import tpu as pltpu
from jax.experimental.pallas import tpu_sc as plsc

_NC = 2
_NS = 16
_NW = _NC * _NS
_CH = 80


def _edge_body(m_ref, rbf_ref, w_ref, x_ref):
    mlp_rbf = jnp.dot(rbf_ref[...], w_ref[...],
                      preferred_element_type=jnp.float32,
                      precision=lax.Precision.HIGHEST)
    x_ref[...] = m_ref[...] * mlp_rbf


def _edge_stage(m, rbf, W_rbf, be=3200):
    e, d = m.shape
    r = rbf.shape[1]
    return pl.pallas_call(
        _edge_body,
        grid=(e // be,),
        in_specs=[
            pl.BlockSpec((be, d), lambda i: (i, 0)),
            pl.BlockSpec((be, r), lambda i: (i, 0)),
            pl.BlockSpec((r, d), lambda i: (0, 0)),
        ],
        out_specs=pl.BlockSpec((be, d), lambda i: (i, 0)),
        out_shape=jax.ShapeDtypeStruct((e, d), jnp.float32),
    )(m, rbf, W_rbf)


def _sc_segment_sum(x, id_j, n_atoms):
    e, d = x.shape
    epw = e // _NW
    nch = epw // _CH
    rpt = (n_atoms // _NS) // 8 * 8
    tail = n_atoms - (_NS - 1) * rpt
    idj3 = id_j.astype(jnp.int32).reshape(_NW, nch, _CH)
    zeros = jnp.zeros((n_atoms, d), jnp.float32)
    mesh = plsc.VectorSubcoreMesh(core_axis_name="c", subcore_axis_name="s")

    @functools.partial(
        pl.kernel,
        out_type=jax.ShapeDtypeStruct((_NC, n_atoms, d), jnp.float32),
        mesh=mesh,
        scratch_types=[
            pltpu.VMEM((nch, _CH), jnp.int32),
            pltpu.VMEM((_CH, d), jnp.float32),
            pltpu.VMEM_SHARED((n_atoms, d), jnp.float32),
        ],
    )
    def sc_kernel(x_hbm, idj_hbm, z_hbm, out_hbm, idx_v, rows_v, acc_sh):
        c = lax.axis_index("c")
        s = lax.axis_index("s")
        wid = s * _NC + c

        @pl.when(s < _NS - 1)
        def _():
            pltpu.sync_copy(z_hbm.at[pl.ds(s * rpt, rpt)],
                            acc_sh.at[pl.ds(s * rpt, rpt)])

        @pl.when(s == _NS - 1)
        def _():
            pltpu.sync_copy(z_hbm.at[pl.ds((_NS - 1) * rpt, tail)],
                            acc_sh.at[pl.ds((_NS - 1) * rpt, tail)])

        pltpu.sync_copy(idj_hbm.at[wid], idx_v)
        plsc.subcore_barrier()

        def body(i, carry):
            pltpu.sync_copy(x_hbm.at[pl.ds(wid * epw + i * _CH, _CH)], rows_v)
            pltpu.sync_copy(rows_v, acc_sh.at[idx_v.at[i]], add=True)
            return carry

        lax.fori_loop(0, nch, body, 0)
        plsc.subcore_barrier()

        @pl.when(s < _NS - 1)
        def _():
            pltpu.sync_copy(acc_sh.at[pl.ds(s * rpt, rpt)],
                            out_hbm.at[c, pl.ds(s * rpt, rpt)])

        @pl.when(s == _NS - 1)
        def _():
            pltpu.sync_copy(acc_sh.at[pl.ds((_NS - 1) * rpt, tail)],
                            out_hbm.at[c, pl.ds((_NS - 1) * rpt, tail)])

    return sc_kernel(x, idj3, zeros)


def _mlp_body(p_ref, w1_ref, a0_ref, a1_ref, b0_ref, b1_ref, out_ref):
    scale = 1.0 / 0.6
    inv_sqrt2 = 0.7071067811865476

    def mm(a, w_ref):
        return jnp.dot(a, w_ref[...], preferred_element_type=jnp.float32,
                       precision=lax.Precision.HIGHEST)

    def ssilu(v):
        return v * jax.nn.sigmoid(v) * scale

    x = ssilu(mm(p_ref[0] + p_ref[1], w1_ref))
    y = ssilu(mm(ssilu(mm(x, a0_ref)), a1_ref))
    x = (x + y) * inv_sqrt2
    y = ssilu(mm(ssilu(mm(x, b0_ref)), b1_ref))
    x = (x + y) * inv_sqrt2
    out_ref[...] = x


def _mlp_stage(parts, W1, res_Ws, bn=2000):
    _, n, d = parts.shape
    wspec = pl.BlockSpec((d, d), lambda i: (0, 0))
    return pl.pallas_call(
        _mlp_body,
        grid=(n // bn,),
        in_specs=[pl.BlockSpec((_NC, bn, d), lambda i: (0, i, 0)),
                  wspec, wspec, wspec, wspec, wspec],
        out_specs=pl.BlockSpec((bn, d), lambda i: (i, 0)),
        out_shape=jax.ShapeDtypeStruct((n, d), jnp.float32),
    )(parts, W1, *res_Ws)


def kernel(h, m, rbf, id_j, W_rbf, W1, res_Ws):
    n_atoms = h.shape[0]
    x = _edge_stage(m, rbf, W_rbf)
    parts = _sc_segment_sum(x, id_j, n_atoms)
    return _mlp_stage(parts, W1, res_Ws)


# speedup vs baseline: 2.3608x; 1.1156x over previous
import functools

import jax
import jax.numpy as jnp
from jax import lax
from jax.experimental import pallas as pl
from jax.experimental.pallas import tpu as pltpu
from jax.experimental.pallas import tpu_sc as plsc

_NC = 2
_NS = 16
_NW = _NC * _NS
_CH = 128
_NBUF = 2


def _edge_body(m_ref, rbf_ref, w_ref, x_ref):
    mlp_rbf = jnp.dot(rbf_ref[...], w_ref[...],
                      preferred_element_type=jnp.float32,
                      precision=lax.Precision.HIGHEST)
    x_ref[...] = m_ref[...] * mlp_rbf


def _edge_stage(m, rbf, W_rbf, be=3200):
    e, d = m.shape
    r = rbf.shape[1]
    return pl.pallas_call(
        _edge_body,
        grid=(e // be,),
        in_specs=[
            pl.BlockSpec((be, d), lambda i: (i, 0)),
            pl.BlockSpec((be, r), lambda i: (i, 0)),
            pl.BlockSpec((r, d), lambda i: (0, 0)),
        ],
        out_specs=pl.BlockSpec((be, d), lambda i: (i, 0)),
        out_shape=jax.ShapeDtypeStruct((e, d), jnp.float32),
    )(m, rbf, W_rbf)


def _sc_segment_sum(x, id_j, n_atoms):
    e, d = x.shape
    nrow = e // _CH
    rpw = -(-nrow // _NW) // 8 * 8 + 8
    tail_rows = nrow - (_NW - 1) * rpw
    assert tail_rows > 0 and rpw % _NBUF == 0 and tail_rows % _NBUF == 0
    rpt = (n_atoms // _NS) // 8 * 8
    tail = n_atoms - (_NS - 1) * rpt
    idj2 = id_j.astype(jnp.int32).reshape(nrow, _CH)
    idj2 = jnp.pad(idj2, ((0, _NW * rpw - nrow), (0, 0)))
    zeros = jnp.zeros((n_atoms, d), jnp.float32)
    mesh = plsc.VectorSubcoreMesh(core_axis_name="c", subcore_axis_name="s")

    @functools.partial(
        pl.kernel,
        out_type=jax.ShapeDtypeStruct((_NC, n_atoms, d), jnp.float32),
        mesh=mesh,
        scratch_types=[
            pltpu.VMEM((rpw, _CH), jnp.int32),
            [pltpu.VMEM((_CH, d), jnp.float32) for _ in range(_NBUF)],
            [pltpu.SemaphoreType.DMA for _ in range(_NBUF)],
            [pltpu.SemaphoreType.DMA for _ in range(_NBUF)],
            pltpu.VMEM_SHARED((n_atoms, d), jnp.float32),
        ],
    )
    def sc_kernel(x_hbm, idj_hbm, z_hbm, out_hbm,
                  idx_v, rows, sg, ss, acc_sh):
        c = lax.axis_index("c")
        s = lax.axis_index("s")
        wid = s * _NC + c
        base_row = wid * rpw
        niter = jnp.where(wid == _NW - 1, tail_rows, rpw) // _NBUF

        @pl.when(s < _NS - 1)
        def _():
            pltpu.sync_copy(z_hbm.at[pl.ds(s * rpt, rpt)],
                            acc_sh.at[pl.ds(s * rpt, rpt)])

        @pl.when(s == _NS - 1)
        def _():
            pltpu.sync_copy(z_hbm.at[pl.ds((_NS - 1) * rpt, tail)],
                            acc_sh.at[pl.ds((_NS - 1) * rpt, tail)])

        pltpu.sync_copy(idj_hbm.at[pl.ds(base_row, rpw)], idx_v)

        plsc.subcore_barrier()

        def body(k, carry):
            r0 = base_row + k * _NBUF
            gd = [pltpu.async_copy(x_hbm.at[pl.ds((r0 + b) * _CH, _CH)],
                                   rows[b], sg[b])
                  for b in range(_NBUF)]
            sd = []
            for b in range(_NBUF):
                gd[b].wait()
                sd.append(pltpu.async_copy(
                    rows[b], acc_sh.at[idx_v.at[k * _NBUF + b]], ss[b],
                    add=True))
            for dsc in sd:
                dsc.wait()
            return carry

        lax.fori_loop(0, niter, body, 0)
        plsc.subcore_barrier()

        @pl.when(s < _NS - 1)
        def _():
            pltpu.sync_copy(acc_sh.at[pl.ds(s * rpt, rpt)],
                            out_hbm.at[c, pl.ds(s * rpt, rpt)])

        @pl.when(s == _NS - 1)
        def _():
            pltpu.sync_copy(acc_sh.at[pl.ds((_NS - 1) * rpt, tail)],
                            out_hbm.at[c, pl.ds((_NS - 1) * rpt, tail)])

    return sc_kernel(x, idj2, zeros)


def _mlp_body(p_ref, w1_ref, a0_ref, a1_ref, b0_ref, b1_ref, out_ref):
    scale = 1.0 / 0.6
    inv_sqrt2 = 0.7071067811865476

    def mm(a, w_ref):
        return jnp.dot(a, w_ref[...], preferred_element_type=jnp.float32,
                       precision=lax.Precision.HIGHEST)

    def ssilu(v):
        return v * jax.nn.sigmoid(v) * scale

    x = ssilu(mm(p_ref[0] + p_ref[1], w1_ref))
    y = ssilu(mm(ssilu(mm(x, a0_ref)), a1_ref))
    x = (x + y) * inv_sqrt2
    y = ssilu(mm(ssilu(mm(x, b0_ref)), b1_ref))
    x = (x + y) * inv_sqrt2
    out_ref[...] = x


def _mlp_stage(parts, W1, res_Ws, bn=2000):
    _, n, d = parts.shape
    wspec = pl.BlockSpec((d, d), lambda i: (0, 0))
    return pl.pallas_call(
        _mlp_body,
        grid=(n // bn,),
        in_specs=[pl.BlockSpec((_NC, bn, d), lambda i: (0, i, 0)),
                  wspec, wspec, wspec, wspec, wspec],
        out_specs=pl.BlockSpec((bn, d), lambda i: (i, 0)),
        out_shape=jax.ShapeDtypeStruct((n, d), jnp.float32),
    )(parts, W1, *res_Ws)


def kernel(h, m, rbf, id_j, W_rbf, W1, res_Ws):
    n_atoms = h.shape[0]
    x = _edge_stage(m, rbf, W_rbf)
    parts = _sc_segment_sum(x, id_j, n_atoms)
    return _mlp_stage(parts, W1, res_Ws)


# speedup vs baseline: 2.9691x; 1.2577x over previous
import functools

import jax
import jax.numpy as jnp
from jax import lax
from jax.experimental import pallas as pl
from jax.experimental.pallas import tpu as pltpu
from jax.experimental.pallas import tpu_sc as plsc

_NC = 2
_NS = 16
_NW = _NC * _NS
_CH = 128
_NBUF = 2


def _edge_body(m_ref, rbft_ref, w_ref, x_ref):
    mlp_rbf = lax.dot_general(rbft_ref[...], w_ref[...],
                              (((0,), (0,)), ((), ())),
                              preferred_element_type=jnp.float32,
                              precision=lax.Precision.HIGHEST)
    x_ref[...] = m_ref[...] * mlp_rbf


def _edge_stage(m, rbf, W_rbf, be=3200):
    e, d = m.shape
    r = rbf.shape[1]
    rbf_t = rbf.T
    return pl.pallas_call(
        _edge_body,
        grid=(e // be,),
        in_specs=[
            pl.BlockSpec((be, d), lambda i: (i, 0)),
            pl.BlockSpec((r, be), lambda i: (0, i)),
            pl.BlockSpec((r, d), lambda i: (0, 0)),
        ],
        out_specs=pl.BlockSpec((be, d), lambda i: (i, 0)),
        out_shape=jax.ShapeDtypeStruct((e, d), jnp.float32),
    )(m, rbf_t, W_rbf)


def _sc_segment_sum(x, id_j, n_atoms):
    e, d = x.shape
    nrow = e // _CH
    rpw = -(-nrow // _NW) // 8 * 8 + 8
    tail_rows = nrow - (_NW - 1) * rpw
    assert tail_rows > 0 and rpw % _NBUF == 0 and tail_rows % _NBUF == 0
    rpt = (n_atoms // _NS) // 8 * 8
    tail = n_atoms - (_NS - 1) * rpt
    idj2 = id_j.astype(jnp.int32).reshape(nrow, _CH)
    idj2 = jnp.pad(idj2, ((0, _NW * rpw - nrow), (0, 0)))
    zeros = jnp.zeros((n_atoms, d), jnp.float32)
    mesh = plsc.VectorSubcoreMesh(core_axis_name="c", subcore_axis_name="s")

    @functools.partial(
        pl.kernel,
        out_type=jax.ShapeDtypeStruct((_NC, n_atoms, d), jnp.float32),
        mesh=mesh,
        scratch_types=[
            pltpu.VMEM((rpw, _CH), jnp.int32),
            [pltpu.VMEM((_CH, d), jnp.float32) for _ in range(_NBUF)],
            [pltpu.SemaphoreType.DMA for _ in range(_NBUF)],
            [pltpu.SemaphoreType.DMA for _ in range(_NBUF)],
            pltpu.VMEM_SHARED((n_atoms, d), jnp.float32),
        ],
    )
    def sc_kernel(x_hbm, idj_hbm, z_hbm, out_hbm,
                  idx_v, rows, sg, ss, acc_sh):
        c = lax.axis_index("c")
        s = lax.axis_index("s")
        wid = s * _NC + c
        base_row = wid * rpw
        niter = jnp.where(wid == _NW - 1, tail_rows, rpw) // _NBUF

        @pl.when(s < _NS - 1)
        def _():
            pltpu.sync_copy(z_hbm.at[pl.ds(s * rpt, rpt)],
                            acc_sh.at[pl.ds(s * rpt, rpt)])

        @pl.when(s == _NS - 1)
        def _():
            pltpu.sync_copy(z_hbm.at[pl.ds((_NS - 1) * rpt, tail)],
                            acc_sh.at[pl.ds((_NS - 1) * rpt, tail)])

        pltpu.sync_copy(idj_hbm.at[pl.ds(base_row, rpw)], idx_v)

        plsc.subcore_barrier()

        def body(k, carry):
            r0 = base_row + k * _NBUF
            gd = [pltpu.async_copy(x_hbm.at[pl.ds((r0 + b) * _CH, _CH)],
                                   rows[b], sg[b])
                  for b in range(_NBUF)]
            sd = []
            for b in range(_NBUF):
                gd[b].wait()
                sd.append(pltpu.async_copy(
                    rows[b], acc_sh.at[idx_v.at[k * _NBUF + b]], ss[b],
                    add=True))
            for dsc in sd:
                dsc.wait()
            return carry

        lax.fori_loop(0, niter, body, 0)
        plsc.subcore_barrier()

        @pl.when(s < _NS - 1)
        def _():
            pltpu.sync_copy(acc_sh.at[pl.ds(s * rpt, rpt)],
                            out_hbm.at[c, pl.ds(s * rpt, rpt)])

        @pl.when(s == _NS - 1)
        def _():
            pltpu.sync_copy(acc_sh.at[pl.ds((_NS - 1) * rpt, tail)],
                            out_hbm.at[c, pl.ds((_NS - 1) * rpt, tail)])

    return sc_kernel(x, idj2, zeros)


def _mlp_body(p_ref, w1_ref, a0_ref, a1_ref, b0_ref, b1_ref, out_ref):
    scale = 1.0 / 0.6
    inv_sqrt2 = 0.7071067811865476

    def mm(a, w_ref):
        return jnp.dot(a, w_ref[...], preferred_element_type=jnp.float32,
                       precision=lax.Precision.HIGHEST)

    def ssilu(v):
        return v * jax.nn.sigmoid(v) * scale

    x = ssilu(mm(p_ref[0] + p_ref[1], w1_ref))
    y = ssilu(mm(ssilu(mm(x, a0_ref)), a1_ref))
    x = (x + y) * inv_sqrt2
    y = ssilu(mm(ssilu(mm(x, b0_ref)), b1_ref))
    x = (x + y) * inv_sqrt2
    out_ref[...] = x


def _mlp_stage(parts, W1, res_Ws, bn=2000):
    _, n, d = parts.shape
    wspec = pl.BlockSpec((d, d), lambda i: (0, 0))
    return pl.pallas_call(
        _mlp_body,
        grid=(n // bn,),
        in_specs=[pl.BlockSpec((_NC, bn, d), lambda i: (0, i, 0)),
                  wspec, wspec, wspec, wspec, wspec],
        out_specs=pl.BlockSpec((bn, d), lambda i: (i, 0)),
        out_shape=jax.ShapeDtypeStruct((n, d), jnp.float32),
    )(parts, W1, *res_Ws)


def kernel(h, m, rbf, id_j, W_rbf, W1, res_Ws):
    n_atoms = h.shape[0]
    x = _edge_stage(m, rbf, W_rbf)
    parts = _sc_segment_sum(x, id_j, n_atoms)
    return _mlp_stage(parts, W1, res_Ws)


# speedup vs baseline: 3.3040x; 1.1128x over previous
import functools

import jax
import jax.numpy as jnp
from jax import lax
from jax.experimental import pallas as pl
from jax.experimental.pallas import tpu as pltpu
from jax.experimental.pallas import tpu_sc as plsc

_NC = 2
_NS = 16
_NW = _NC * _NS
_CH = 128
_NBUF = 2


def _edge_body(m_ref, rbft_ref, w_ref, x_ref):
    mlp_rbf = lax.dot_general(rbft_ref[...], w_ref[...],
                              (((0,), (0,)), ((), ())),
                              preferred_element_type=jnp.float32,
                              precision=lax.Precision.DEFAULT)
    x_ref[...] = m_ref[...] * mlp_rbf


def _edge_stage(m, rbf, W_rbf, be=3200):
    e, d = m.shape
    r = rbf.shape[1]
    rbf_t = rbf.T
    return pl.pallas_call(
        _edge_body,
        grid=(e // be,),
        in_specs=[
            pl.BlockSpec((be, d), lambda i: (i, 0)),
            pl.BlockSpec((r, be), lambda i: (0, i)),
            pl.BlockSpec((r, d), lambda i: (0, 0)),
        ],
        out_specs=pl.BlockSpec((be, d), lambda i: (i, 0)),
        out_shape=jax.ShapeDtypeStruct((e, d), jnp.float32),
    )(m, rbf_t, W_rbf)


def _sc_segment_sum(x, id_j, n_atoms):
    e, d = x.shape
    nrow = e // _CH
    rpw = -(-nrow // _NW) // 8 * 8 + 8
    tail_rows = nrow - (_NW - 1) * rpw
    assert tail_rows > 0 and rpw % _NBUF == 0 and tail_rows % _NBUF == 0
    rpt = (n_atoms // _NS) // 8 * 8
    tail = n_atoms - (_NS - 1) * rpt
    idj2 = id_j.astype(jnp.int32).reshape(nrow, _CH)
    idj2 = jnp.pad(idj2, ((0, _NW * rpw - nrow), (0, 0)))
    zeros = jnp.zeros((n_atoms, d), jnp.float32)
    mesh = plsc.VectorSubcoreMesh(core_axis_name="c", subcore_axis_name="s")

    @functools.partial(
        pl.kernel,
        out_type=jax.ShapeDtypeStruct((_NC, n_atoms, d), jnp.float32),
        mesh=mesh,
        scratch_types=[
            pltpu.VMEM((rpw, _CH), jnp.int32),
            [pltpu.VMEM((_CH, d), jnp.float32) for _ in range(_NBUF)],
            [pltpu.SemaphoreType.DMA for _ in range(_NBUF)],
            [pltpu.SemaphoreType.DMA for _ in range(_NBUF)],
            pltpu.VMEM_SHARED((n_atoms, d), jnp.float32),
        ],
    )
    def sc_kernel(x_hbm, idj_hbm, z_hbm, out_hbm,
                  idx_v, rows, sg, ss, acc_sh):
        c = lax.axis_index("c")
        s = lax.axis_index("s")
        wid = s * _NC + c
        base_row = wid * rpw
        niter = jnp.where(wid == _NW - 1, tail_rows, rpw) // _NBUF

        @pl.when(s < _NS - 1)
        def _():
            pltpu.sync_copy(z_hbm.at[pl.ds(s * rpt, rpt)],
                            acc_sh.at[pl.ds(s * rpt, rpt)])

        @pl.when(s == _NS - 1)
        def _():
            pltpu.sync_copy(z_hbm.at[pl.ds((_NS - 1) * rpt, tail)],
                            acc_sh.at[pl.ds((_NS - 1) * rpt, tail)])

        pltpu.sync_copy(idj_hbm.at[pl.ds(base_row, rpw)], idx_v)

        plsc.subcore_barrier()

        def body(k, carry):
            r0 = base_row + k * _NBUF
            gd = [pltpu.async_copy(x_hbm.at[pl.ds((r0 + b) * _CH, _CH)],
                                   rows[b], sg[b])
                  for b in range(_NBUF)]
            sd = []
            for b in range(_NBUF):
                gd[b].wait()
                sd.append(pltpu.async_copy(
                    rows[b], acc_sh.at[idx_v.at[k * _NBUF + b]], ss[b],
                    add=True))
            for dsc in sd:
                dsc.wait()
            return carry

        lax.fori_loop(0, niter, body, 0)
        plsc.subcore_barrier()

        @pl.when(s < _NS - 1)
        def _():
            pltpu.sync_copy(acc_sh.at[pl.ds(s * rpt, rpt)],
                            out_hbm.at[c, pl.ds(s * rpt, rpt)])

        @pl.when(s == _NS - 1)
        def _():
            pltpu.sync_copy(acc_sh.at[pl.ds((_NS - 1) * rpt, tail)],
                            out_hbm.at[c, pl.ds((_NS - 1) * rpt, tail)])

    return sc_kernel(x, idj2, zeros)


def _mlp_body(p_ref, w1_ref, a0_ref, a1_ref, b0_ref, b1_ref, out_ref):
    scale = 1.0 / 0.6
    inv_sqrt2 = 0.7071067811865476

    def mm(a, w_ref):
        return jnp.dot(a, w_ref[...], preferred_element_type=jnp.float32,
                       precision=lax.Precision.HIGHEST)

    def ssilu(v):
        return v * jax.nn.sigmoid(v) * scale

    x = ssilu(mm(p_ref[0] + p_ref[1], w1_ref))
    y = ssilu(mm(ssilu(mm(x, a0_ref)), a1_ref))
    x = (x + y) * inv_sqrt2
    y = ssilu(mm(ssilu(mm(x, b0_ref)), b1_ref))
    x = (x + y) * inv_sqrt2
    out_ref[...] = x


def _mlp_stage(parts, W1, res_Ws, bn=2000):
    _, n, d = parts.shape
    wspec = pl.BlockSpec((d, d), lambda i: (0, 0))
    return pl.pallas_call(
        _mlp_body,
        grid=(n // bn,),
        in_specs=[pl.BlockSpec((_NC, bn, d), lambda i: (0, i, 0)),
                  wspec, wspec, wspec, wspec, wspec],
        out_specs=pl.BlockSpec((bn, d), lambda i: (i, 0)),
        out_shape=jax.ShapeDtypeStruct((n, d), jnp.float32),
    )(parts, W1, *res_Ws)


def kernel(h, m, rbf, id_j, W_rbf, W1, res_Ws):
    n_atoms = h.shape[0]
    x = _edge_stage(m, rbf, W_rbf)
    parts = _sc_segment_sum(x, id_j, n_atoms)
    return _mlp_stage(parts, W1, res_Ws)


# speedup vs baseline: 3.5940x; 1.0878x over previous
import functools

import jax
import jax.numpy as jnp
from jax import lax
from jax.experimental import pallas as pl
from jax.experimental.pallas import tpu as pltpu
from jax.experimental.pallas import tpu_sc as plsc

_NC = 2
_NS = 16
_NW = _NC * _NS
_CH = 128
_NBUF = 2


def _edge_body(m_ref, rbft_ref, w_ref, x_ref):
    mlp_rbf = lax.dot_general(rbft_ref[...], w_ref[...],
                              (((0,), (0,)), ((), ())),
                              preferred_element_type=jnp.float32,
                              precision=lax.Precision.DEFAULT)
    x_ref[...] = m_ref[...] * mlp_rbf


def _edge_stage(m, rbf, W_rbf, be=3200):
    e, d = m.shape
    r = rbf.shape[1]
    rbf_t = rbf.T
    return pl.pallas_call(
        _edge_body,
        grid=(e // be,),
        in_specs=[
            pl.BlockSpec((be, d), lambda i: (i, 0)),
            pl.BlockSpec((r, be), lambda i: (0, i)),
            pl.BlockSpec((r, d), lambda i: (0, 0)),
        ],
        out_specs=pl.BlockSpec((be, d), lambda i: (i, 0)),
        out_shape=jax.ShapeDtypeStruct((e, d), jnp.float32),
    )(m, rbf_t, W_rbf)


def _sc_segment_sum(x, id_j, n_atoms):
    e, d = x.shape
    nrow = e // _CH
    rpw = -(-nrow // _NW) // 8 * 8 + 8
    tail_rows = nrow - (_NW - 1) * rpw
    assert tail_rows > 0 and rpw % _NBUF == 0 and tail_rows % _NBUF == 0
    rpt = (n_atoms // _NS) // 8 * 8
    tail = n_atoms - (_NS - 1) * rpt
    idj2 = id_j.astype(jnp.int32).reshape(nrow, _CH)
    idj2 = jnp.pad(idj2, ((0, _NW * rpw - nrow), (0, 0)))
    zeros = jnp.zeros((n_atoms, d), jnp.float32)
    mesh = plsc.VectorSubcoreMesh(core_axis_name="c", subcore_axis_name="s")

    @functools.partial(
        pl.kernel,
        out_type=jax.ShapeDtypeStruct((_NC, n_atoms, d), jnp.float32),
        mesh=mesh,
        scratch_types=[
            pltpu.VMEM((rpw, _CH), jnp.int32),
            [pltpu.VMEM((_CH, d), jnp.float32) for _ in range(_NBUF)],
            [pltpu.SemaphoreType.DMA for _ in range(_NBUF)],
            [pltpu.SemaphoreType.DMA for _ in range(_NBUF)],
            pltpu.VMEM_SHARED((n_atoms, d), jnp.float32),
        ],
    )
    def sc_kernel(x_hbm, idj_hbm, z_hbm, out_hbm,
                  idx_v, rows, sg, ss, acc_sh):
        c = lax.axis_index("c")
        s = lax.axis_index("s")
        wid = s * _NC + c
        base_row = wid * rpw
        niter = jnp.where(wid == _NW - 1, tail_rows, rpw) // _NBUF

        @pl.when(s < _NS - 1)
        def _():
            pltpu.sync_copy(z_hbm.at[pl.ds(s * rpt, rpt)],
                            acc_sh.at[pl.ds(s * rpt, rpt)])

        @pl.when(s == _NS - 1)
        def _():
            pltpu.sync_copy(z_hbm.at[pl.ds((_NS - 1) * rpt, tail)],
                            acc_sh.at[pl.ds((_NS - 1) * rpt, tail)])

        pltpu.sync_copy(idj_hbm.at[pl.ds(base_row, rpw)], idx_v)

        plsc.subcore_barrier()

        def body(k, carry):
            r0 = base_row + k * _NBUF
            gd = [pltpu.async_copy(x_hbm.at[pl.ds((r0 + b) * _CH, _CH)],
                                   rows[b], sg[b])
                  for b in range(_NBUF)]
            sd = []
            for b in range(_NBUF):
                gd[b].wait()
                sd.append(pltpu.async_copy(
                    rows[b], acc_sh.at[idx_v.at[k * _NBUF + b]], ss[b],
                    add=True))
            for dsc in sd:
                dsc.wait()
            return carry

        lax.fori_loop(0, niter, body, 0)
        plsc.subcore_barrier()

        @pl.when(s < _NS - 1)
        def _():
            pltpu.sync_copy(acc_sh.at[pl.ds(s * rpt, rpt)],
                            out_hbm.at[c, pl.ds(s * rpt, rpt)])

        @pl.when(s == _NS - 1)
        def _():
            pltpu.sync_copy(acc_sh.at[pl.ds((_NS - 1) * rpt, tail)],
                            out_hbm.at[c, pl.ds((_NS - 1) * rpt, tail)])

    return sc_kernel(x, idj2, zeros)


def _mlp_body(p_ref, w1_ref, a0_ref, a1_ref, b0_ref, b1_ref, out_ref):
    scale = 1.0 / 0.6
    inv_sqrt2 = 0.7071067811865476

    def mm(a, w_ref):
        return jnp.dot(a, w_ref[...], preferred_element_type=jnp.float32,
                       precision=lax.Precision.DEFAULT)

    def ssilu(v):
        return v * jax.nn.sigmoid(v) * scale

    x = ssilu(mm(p_ref[0] + p_ref[1], w1_ref))
    y = ssilu(mm(ssilu(mm(x, a0_ref)), a1_ref))
    x = (x + y) * inv_sqrt2
    y = ssilu(mm(ssilu(mm(x, b0_ref)), b1_ref))
    x = (x + y) * inv_sqrt2
    out_ref[...] = x


def _mlp_stage(parts, W1, res_Ws, bn=2000):
    _, n, d = parts.shape
    wspec = pl.BlockSpec((d, d), lambda i: (0, 0))
    return pl.pallas_call(
        _mlp_body,
        grid=(n // bn,),
        in_specs=[pl.BlockSpec((_NC, bn, d), lambda i: (0, i, 0)),
                  wspec, wspec, wspec, wspec, wspec],
        out_specs=pl.BlockSpec((bn, d), lambda i: (i, 0)),
        out_shape=jax.ShapeDtypeStruct((n, d), jnp.float32),
    )(parts, W1, *res_Ws)


def kernel(h, m, rbf, id_j, W_rbf, W1, res_Ws):
    n_atoms = h.shape[0]
    x = _edge_stage(m, rbf, W_rbf)
    parts = _sc_segment_sum(x, id_j, n_atoms)
    return _mlp_stage(parts, W1, res_Ws)


# speedup vs baseline: 4.0951x; 1.1394x over previous
import functools

import jax
import jax.numpy as jnp
from jax import lax
from jax.experimental import pallas as pl
from jax.experimental.pallas import tpu as pltpu
from jax.experimental.pallas import tpu_sc as plsc

_NC = 2
_NS = 16
_NW = _NC * _NS
_CH = 128
_NBUF = 2


def _edge_body(m_ref, rbft_ref, w_ref, x_ref):
    mlp_rbf = lax.dot_general(rbft_ref[...], w_ref[...],
                              (((0,), (0,)), ((), ())),
                              preferred_element_type=jnp.float32,
                              precision=lax.Precision.DEFAULT)
    x_ref[...] = m_ref[...] * mlp_rbf


def _edge_stage(m, rbf_t, W_rbf, row0, nrows, be=3200):
    d = m.shape[1]
    r = rbf_t.shape[0]
    blk0 = row0 // be
    return pl.pallas_call(
        _edge_body,
        grid=(nrows // be,),
        in_specs=[
            pl.BlockSpec((be, d), lambda i: (i + blk0, 0)),
            pl.BlockSpec((r, be), lambda i: (0, i + blk0)),
            pl.BlockSpec((r, d), lambda i: (0, 0)),
        ],
        out_specs=pl.BlockSpec((be, d), lambda i: (i, 0)),
        out_shape=jax.ShapeDtypeStruct((nrows, d), jnp.float32),
    )(m, rbf_t, W_rbf)


def _sc_segment_sum(x, id_j, n_atoms):
    e, d = x.shape
    nrow = e // _CH
    rpw = (-(-nrow // _NW) + 7) // 8 * 8
    tail_rows = nrow - (_NW - 1) * rpw
    assert tail_rows > 0 and rpw % _NBUF == 0 and tail_rows % _NBUF == 0
    rpt = (n_atoms // _NS) // 8 * 8
    tail = n_atoms - (_NS - 1) * rpt
    idj2 = id_j.astype(jnp.int32).reshape(nrow, _CH)
    idj2 = jnp.pad(idj2, ((0, _NW * rpw - nrow), (0, 0)))
    zeros = jnp.zeros((n_atoms, d), jnp.float32)
    mesh = plsc.VectorSubcoreMesh(core_axis_name="c", subcore_axis_name="s")

    @functools.partial(
        pl.kernel,
        out_type=jax.ShapeDtypeStruct((_NC, n_atoms, d), jnp.float32),
        mesh=mesh,
        scratch_types=[
            pltpu.VMEM((rpw, _CH), jnp.int32),
            [pltpu.VMEM((_CH, d), jnp.float32) for _ in range(_NBUF)],
            [pltpu.SemaphoreType.DMA for _ in range(_NBUF)],
            [pltpu.SemaphoreType.DMA for _ in range(_NBUF)],
            pltpu.VMEM_SHARED((n_atoms, d), jnp.float32),
        ],
    )
    def sc_kernel(x_hbm, idj_hbm, z_hbm, out_hbm,
                  idx_v, rows, sg, ss, acc_sh):
        c = lax.axis_index("c")
        s = lax.axis_index("s")
        wid = s * _NC + c
        base_row = wid * rpw
        niter = jnp.where(wid == _NW - 1, tail_rows, rpw) // _NBUF

        @pl.when(s < _NS - 1)
        def _():
            pltpu.sync_copy(z_hbm.at[pl.ds(s * rpt, rpt)],
                            acc_sh.at[pl.ds(s * rpt, rpt)])

        @pl.when(s == _NS - 1)
        def _():
            pltpu.sync_copy(z_hbm.at[pl.ds((_NS - 1) * rpt, tail)],
                            acc_sh.at[pl.ds((_NS - 1) * rpt, tail)])

        pltpu.sync_copy(idj_hbm.at[pl.ds(base_row, rpw)], idx_v)

        plsc.subcore_barrier()

        def body(k, carry):
            r0 = base_row + k * _NBUF
            gd = [pltpu.async_copy(x_hbm.at[pl.ds((r0 + b) * _CH, _CH)],
                                   rows[b], sg[b])
                  for b in range(_NBUF)]
            sd = []
            for b in range(_NBUF):
                gd[b].wait()
                sd.append(pltpu.async_copy(
                    rows[b], acc_sh.at[idx_v.at[k * _NBUF + b]], ss[b],
                    add=True))
            for dsc in sd:
                dsc.wait()
            return carry

        lax.fori_loop(0, niter, body, 0)
        plsc.subcore_barrier()

        @pl.when(s < _NS - 1)
        def _():
            pltpu.sync_copy(acc_sh.at[pl.ds(s * rpt, rpt)],
                            out_hbm.at[c, pl.ds(s * rpt, rpt)])

        @pl.when(s == _NS - 1)
        def _():
            pltpu.sync_copy(acc_sh.at[pl.ds((_NS - 1) * rpt, tail)],
                            out_hbm.at[c, pl.ds((_NS - 1) * rpt, tail)])

    return sc_kernel(x, idj2, zeros)


def _mlp_body(p1_ref, p2_ref, w1_ref, a0_ref, a1_ref, b0_ref, b1_ref,
              out_ref):
    scale = 1.0 / 0.6
    inv_sqrt2 = 0.7071067811865476

    def mm(a, w_ref):
        return jnp.dot(a, w_ref[...], preferred_element_type=jnp.float32,
                       precision=lax.Precision.DEFAULT)

    def ssilu(v):
        return v * jax.nn.sigmoid(v) * scale

    x2 = (p1_ref[0] + p1_ref[1]) + (p2_ref[0] + p2_ref[1])
    x = ssilu(mm(x2, w1_ref))
    y = ssilu(mm(ssilu(mm(x, a0_ref)), a1_ref))
    x = (x + y) * inv_sqrt2
    y = ssilu(mm(ssilu(mm(x, b0_ref)), b1_ref))
    x = (x + y) * inv_sqrt2
    out_ref[...] = x


def _mlp_stage(parts1, parts2, W1, res_Ws, bn=2000):
    _, n, d = parts1.shape
    pspec = pl.BlockSpec((_NC, bn, d), lambda i: (0, i, 0))
    wspec = pl.BlockSpec((d, d), lambda i: (0, 0))
    return pl.pallas_call(
        _mlp_body,
        grid=(n // bn,),
        in_specs=[pspec, pspec, wspec, wspec, wspec, wspec, wspec],
        out_specs=pl.BlockSpec((bn, d), lambda i: (i, 0)),
        out_shape=jax.ShapeDtypeStruct((n, d), jnp.float32),
    )(parts1, parts2, W1, *res_Ws)


def kernel(h, m, rbf, id_j, W_rbf, W1, res_Ws):
    n_atoms = h.shape[0]
    e = m.shape[0]
    half = e // 2
    rbf_t = rbf.T
    x1 = _edge_stage(m, rbf_t, W_rbf, 0, half)
    parts1 = _sc_segment_sum(x1, id_j[:half], n_atoms)
    x2 = _edge_stage(m, rbf_t, W_rbf, half, half)
    parts2 = _sc_segment_sum(x2, id_j[half:], n_atoms)
    return _mlp_stage(parts1, parts2, W1, res_Ws)


# speedup vs baseline: 4.1821x; 1.0212x over previous
import functools

import jax
import jax.numpy as jnp
from jax import lax
from jax.experimental import pallas as pl
from jax.experimental.pallas import tpu as pltpu
from jax.experimental.pallas import tpu_sc as plsc

_NC = 2
_NS = 16
_NW = _NC * _NS
_CH = 128
_NBUF = 2


def _edge_body(m_ref, rbft_ref, w_ref, x_ref):
    mlp_rbf = lax.dot_general(rbft_ref[...], w_ref[...],
                              (((0,), (0,)), ((), ())),
                              preferred_element_type=jnp.float32,
                              precision=lax.Precision.DEFAULT)
    x_ref[...] = m_ref[...] * mlp_rbf


def _edge_stage(m, rbf_t, W_rbf, row0, nrows, be=3200):
    d = m.shape[1]
    r = rbf_t.shape[0]
    blk0 = row0 // be
    return pl.pallas_call(
        _edge_body,
        grid=(nrows // be,),
        in_specs=[
            pl.BlockSpec((be, d), lambda i: (i + blk0, 0)),
            pl.BlockSpec((r, be), lambda i: (0, i + blk0)),
            pl.BlockSpec((r, d), lambda i: (0, 0)),
        ],
        out_specs=pl.BlockSpec((be, d), lambda i: (i, 0)),
        out_shape=jax.ShapeDtypeStruct((nrows, d), jnp.float32),
    )(m, rbf_t, W_rbf)


def _sc_segment_sum(x, id_j, n_atoms):
    e, d = x.shape
    nrow = e // _CH
    rpw = (-(-nrow // _NW) + 7) // 8 * 8
    tail_rows = nrow - (_NW - 1) * rpw
    assert tail_rows > 0 and rpw % _NBUF == 0 and tail_rows % _NBUF == 0
    rpt = (n_atoms // _NS) // 8 * 8
    tail = n_atoms - (_NS - 1) * rpt
    idj2 = id_j.astype(jnp.int32).reshape(nrow, _CH)
    idj2 = jnp.pad(idj2, ((0, _NW * rpw - nrow), (0, 0)))
    zeros = jnp.zeros((n_atoms, d), jnp.float32)
    mesh = plsc.VectorSubcoreMesh(core_axis_name="c", subcore_axis_name="s")

    @functools.partial(
        pl.kernel,
        out_type=jax.ShapeDtypeStruct((_NC, n_atoms, d), jnp.float32),
        mesh=mesh,
        scratch_types=[
            pltpu.VMEM((rpw, _CH), jnp.int32),
            [pltpu.VMEM((_CH, d), jnp.float32) for _ in range(_NBUF)],
            [pltpu.SemaphoreType.DMA for _ in range(_NBUF)],
            [pltpu.SemaphoreType.DMA for _ in range(_NBUF)],
            pltpu.VMEM_SHARED((n_atoms, d), jnp.float32),
        ],
    )
    def sc_kernel(x_hbm, idj_hbm, z_hbm, out_hbm,
                  idx_v, rows, sg, ss, acc_sh):
        c = lax.axis_index("c")
        s = lax.axis_index("s")
        wid = s * _NC + c
        base_row = wid * rpw
        niter = jnp.where(wid == _NW - 1, tail_rows, rpw) // _NBUF

        @pl.when(s < _NS - 1)
        def _():
            pltpu.sync_copy(z_hbm.at[pl.ds(s * rpt, rpt)],
                            acc_sh.at[pl.ds(s * rpt, rpt)])

        @pl.when(s == _NS - 1)
        def _():
            pltpu.sync_copy(z_hbm.at[pl.ds((_NS - 1) * rpt, tail)],
                            acc_sh.at[pl.ds((_NS - 1) * rpt, tail)])

        pltpu.sync_copy(idj_hbm.at[pl.ds(base_row, rpw)], idx_v)

        nrows_w = niter * _NBUF

        def gather(r, b):
            pltpu.async_copy(x_hbm.at[pl.ds((base_row + r) * _CH, _CH)],
                             rows[b], sg[b])

        for b in range(_NBUF):
            gather(b, b)
        plsc.subcore_barrier()

        def body(k, carry):
            r0 = k * _NBUF
            for b in range(_NBUF):
                pltpu.make_async_copy(
                    x_hbm.at[pl.ds((base_row + r0 + b) * _CH, _CH)],
                    rows[b], sg[b]).wait()
                pltpu.async_copy(rows[b], acc_sh.at[idx_v.at[r0 + b]], ss[b],
                                 add=True)
            for b in range(_NBUF):
                r = r0 + b
                pltpu.make_async_copy(rows[b], acc_sh.at[idx_v.at[r]],
                                      ss[b]).wait()

                @pl.when(r + _NBUF < nrows_w)
                def _():
                    gather(r + _NBUF, b)

            return carry

        lax.fori_loop(0, niter, body, 0)
        plsc.subcore_barrier()

        @pl.when(s < _NS - 1)
        def _():
            pltpu.sync_copy(acc_sh.at[pl.ds(s * rpt, rpt)],
                            out_hbm.at[c, pl.ds(s * rpt, rpt)])

        @pl.when(s == _NS - 1)
        def _():
            pltpu.sync_copy(acc_sh.at[pl.ds((_NS - 1) * rpt, tail)],
                            out_hbm.at[c, pl.ds((_NS - 1) * rpt, tail)])

    return sc_kernel(x, idj2, zeros)


def _mlp_body(p1_ref, p2_ref, w1_ref, a0_ref, a1_ref, b0_ref, b1_ref,
              out_ref):
    scale = 1.0 / 0.6
    inv_sqrt2 = 0.7071067811865476

    def mm(a, w_ref):
        return jnp.dot(a, w_ref[...], preferred_element_type=jnp.float32,
                       precision=lax.Precision.DEFAULT)

    def ssilu(v):
        return v * jax.nn.sigmoid(v) * scale

    x2 = (p1_ref[0] + p1_ref[1]) + (p2_ref[0] + p2_ref[1])
    x = ssilu(mm(x2, w1_ref))
    y = ssilu(mm(ssilu(mm(x, a0_ref)), a1_ref))
    x = (x + y) * inv_sqrt2
    y = ssilu(mm(ssilu(mm(x, b0_ref)), b1_ref))
    x = (x + y) * inv_sqrt2
    out_ref[...] = x


def _mlp_stage(parts1, parts2, W1, res_Ws, bn=2000):
    _, n, d = parts1.shape
    pspec = pl.BlockSpec((_NC, bn, d), lambda i: (0, i, 0))
    wspec = pl.BlockSpec((d, d), lambda i: (0, 0))
    return pl.pallas_call(
        _mlp_body,
        grid=(n // bn,),
        in_specs=[pspec, pspec, wspec, wspec, wspec, wspec, wspec],
        out_specs=pl.BlockSpec((bn, d), lambda i: (i, 0)),
        out_shape=jax.ShapeDtypeStruct((n, d), jnp.float32),
    )(parts1, parts2, W1, *res_Ws)


def kernel(h, m, rbf, id_j, W_rbf, W1, res_Ws):
    n_atoms = h.shape[0]
    e = m.shape[0]
    half = e // 2
    rbf_t = rbf.T
    x1 = _edge_stage(m, rbf_t, W_rbf, 0, half)
    parts1 = _sc_segment_sum(x1, id_j[:half], n_atoms)
    x2 = _edge_stage(m, rbf_t, W_rbf, half, half)
    parts2 = _sc_segment_sum(x2, id_j[half:], n_atoms)
    return _mlp_stage(parts1, parts2, W1, res_Ws)


# speedup vs baseline: 4.2923x; 1.0263x over previous
import functools

import jax
import jax.numpy as jnp
from jax import lax
from jax.experimental import pallas as pl
from jax.experimental.pallas import tpu as pltpu
from jax.experimental.pallas import tpu_sc as plsc

_NC = 2
_NS = 16
_NW = _NC * _NS
_CH = 128
_NBUF = 2


def _edge_body(m_ref, rbft_ref, w_ref, x_ref):
    mlp_rbf = lax.dot_general(rbft_ref[...], w_ref[...],
                              (((0,), (0,)), ((), ())),
                              preferred_element_type=jnp.float32,
                              precision=lax.Precision.DEFAULT)
    x_ref[...] = m_ref[...] * mlp_rbf


def _edge_stage(m, rbf_t, W_rbf, row0, nrows, be=6400):
    d = m.shape[1]
    r = rbf_t.shape[0]
    blk0 = row0 // be
    return pl.pallas_call(
        _edge_body,
        grid=(nrows // be,),
        in_specs=[
            pl.BlockSpec((be, d), lambda i: (i + blk0, 0)),
            pl.BlockSpec((r, be), lambda i: (0, i + blk0)),
            pl.BlockSpec((r, d), lambda i: (0, 0)),
        ],
        out_specs=pl.BlockSpec((be, d), lambda i: (i, 0)),
        out_shape=jax.ShapeDtypeStruct((nrows, d), jnp.float32),
    )(m, rbf_t, W_rbf)


def _sc_segment_sum(x, id_j, n_atoms):
    e, d = x.shape
    nrow = e // _CH
    rpw = (-(-nrow // _NW) + 7) // 8 * 8
    tail_rows = nrow - (_NW - 1) * rpw
    assert tail_rows > 0 and rpw % _NBUF == 0 and tail_rows % _NBUF == 0
    rpt = (n_atoms // _NS) // 8 * 8
    tail = n_atoms - (_NS - 1) * rpt
    idj2 = id_j.astype(jnp.int32).reshape(nrow, _CH)
    idj2 = jnp.pad(idj2, ((0, _NW * rpw - nrow), (0, 0)))
    zeros = jnp.zeros((n_atoms, d), jnp.float32)
    mesh = plsc.VectorSubcoreMesh(core_axis_name="c", subcore_axis_name="s")

    @functools.partial(
        pl.kernel,
        out_type=jax.ShapeDtypeStruct((_NC, n_atoms, d), jnp.float32),
        mesh=mesh,
        scratch_types=[
            pltpu.VMEM((rpw, _CH), jnp.int32),
            [pltpu.VMEM((_CH, d), jnp.float32) for _ in range(_NBUF)],
            [pltpu.SemaphoreType.DMA for _ in range(_NBUF)],
            [pltpu.SemaphoreType.DMA for _ in range(_NBUF)],
            pltpu.VMEM_SHARED((n_atoms, d), jnp.float32),
        ],
    )
    def sc_kernel(x_hbm, idj_hbm, z_hbm, out_hbm,
                  idx_v, rows, sg, ss, acc_sh):
        c = lax.axis_index("c")
        s = lax.axis_index("s")
        wid = s * _NC + c
        base_row = wid * rpw
        niter = jnp.where(wid == _NW - 1, tail_rows, rpw) // _NBUF

        @pl.when(s < _NS - 1)
        def _():
            pltpu.sync_copy(z_hbm.at[pl.ds(s * rpt, rpt)],
                            acc_sh.at[pl.ds(s * rpt, rpt)])

        @pl.when(s == _NS - 1)
        def _():
            pltpu.sync_copy(z_hbm.at[pl.ds((_NS - 1) * rpt, tail)],
                            acc_sh.at[pl.ds((_NS - 1) * rpt, tail)])

        pltpu.sync_copy(idj_hbm.at[pl.ds(base_row, rpw)], idx_v)

        nrows_w = niter * _NBUF

        def gather(r, b):
            pltpu.async_copy(x_hbm.at[pl.ds((base_row + r) * _CH, _CH)],
                             rows[b], sg[b])

        for b in range(_NBUF):
            gather(b, b)
        plsc.subcore_barrier()

        def body(k, carry):
            r0 = k * _NBUF
            for b in range(_NBUF):
                pltpu.make_async_copy(
                    x_hbm.at[pl.ds((base_row + r0 + b) * _CH, _CH)],
                    rows[b], sg[b]).wait()
                pltpu.async_copy(rows[b], acc_sh.at[idx_v.at[r0 + b]], ss[b],
                                 add=True)
            for b in range(_NBUF):
                r = r0 + b
                pltpu.make_async_copy(rows[b], acc_sh.at[idx_v.at[r]],
                                      ss[b]).wait()

                @pl.when(r + _NBUF < nrows_w)
                def _():
                    gather(r + _NBUF, b)

            return carry

        lax.fori_loop(0, niter, body, 0)
        plsc.subcore_barrier()

        @pl.when(s < _NS - 1)
        def _():
            pltpu.sync_copy(acc_sh.at[pl.ds(s * rpt, rpt)],
                            out_hbm.at[c, pl.ds(s * rpt, rpt)])

        @pl.when(s == _NS - 1)
        def _():
            pltpu.sync_copy(acc_sh.at[pl.ds((_NS - 1) * rpt, tail)],
                            out_hbm.at[c, pl.ds((_NS - 1) * rpt, tail)])

    return sc_kernel(x, idj2, zeros)


def _mlp_body(p1_ref, p2_ref, w1_ref, a0_ref, a1_ref, b0_ref, b1_ref,
              out_ref):
    scale = 1.0 / 0.6
    inv_sqrt2 = 0.7071067811865476

    def mm(a, w_ref):
        return jnp.dot(a, w_ref[...], preferred_element_type=jnp.float32,
                       precision=lax.Precision.DEFAULT)

    def ssilu(v):
        return v * jax.nn.sigmoid(v) * scale

    x2 = (p1_ref[0] + p1_ref[1]) + (p2_ref[0] + p2_ref[1])
    x = ssilu(mm(x2, w1_ref))
    y = ssilu(mm(ssilu(mm(x, a0_ref)), a1_ref))
    x = (x + y) * inv_sqrt2
    y = ssilu(mm(ssilu(mm(x, b0_ref)), b1_ref))
    x = (x + y) * inv_sqrt2
    out_ref[...] = x


def _mlp_stage(parts1, parts2, W1, res_Ws, bn=2000):
    _, n, d = parts1.shape
    pspec = pl.BlockSpec((_NC, bn, d), lambda i: (0, i, 0))
    wspec = pl.BlockSpec((d, d), lambda i: (0, 0))
    return pl.pallas_call(
        _mlp_body,
        grid=(n // bn,),
        in_specs=[pspec, pspec, wspec, wspec, wspec, wspec, wspec],
        out_specs=pl.BlockSpec((bn, d), lambda i: (i, 0)),
        out_shape=jax.ShapeDtypeStruct((n, d), jnp.float32),
    )(parts1, parts2, W1, *res_Ws)


def kernel(h, m, rbf, id_j, W_rbf, W1, res_Ws):
    n_atoms = h.shape[0]
    e = m.shape[0]
    half = e // 2
    rbf_t = rbf.T
    x1 = _edge_stage(m, rbf_t, W_rbf, 0, half)
    parts1 = _sc_segment_sum(x1, id_j[:half], n_atoms)
    x2 = _edge_stage(m, rbf_t, W_rbf, half, half)
    parts2 = _sc_segment_sum(x2, id_j[half:], n_atoms)
    return _mlp_stage(parts1, parts2, W1, res_Ws)
